# baseline (speedup 1.0000x reference)
import jax
import jax.numpy as jnp
from jax import lax
from jax.experimental import pallas as pl
from jax.experimental.pallas import tpu as pltpu
from jax.experimental.pallas import tpu_sc as plsc

VOCAB = 1000000
D = 32
B = 16384
L = 200
NC = 2
NS = 16
LANES = 16
NW = NC * NS
TOK = B * L
TOK_PER_TILE = TOK // NW
B_PER_TILE = B // NW
SW = 4 * L
SPP = 16
PANELTOK = SW * SPP
NPANEL = TOK_PER_TILE // PANELTOK
ROWS_PER_PANEL = PANELTOK // L
UNROLL = 8
SCALE = 1.0 / L


def _embed_body(tokens_hbm, table_hbm, out_hbm,
                idx_v, buf0, buf1, outbuf, sem0, sem1):
    c = lax.axis_index("c")
    s = lax.axis_index("s")
    tile = c * NS + s
    tok0 = tile * TOK_PER_TILE
    out_row0 = tile * B_PER_TILE

    vzero = jnp.zeros((LANES,), jnp.float32)

    def start_gather(g, buf, sem):
        pltpu.async_copy(table_hbm.at[idx_v.at[pl.ds(g * SW, SW)]], buf, sem)

    def wait_gather(buf, sem):
        pltpu.make_async_copy(table_hbm.at[idx_v.at[pl.ds(0, SW)]], buf, sem).wait()

    def accumulate(buf, row0):
        for q in range(SW // L):
            def body(i, carry):
                a0, a1 = carry
                for u in range(UNROLL):
                    r = q * L + i * UNROLL + u
                    bv = buf[r, pl.ds(0, 2 * LANES)]
                    u0, u1 = plsc.unpack(bv, format=plsc.PackFormat.INTERLEAVED)
                    a0 = a0 + u0
                    a1 = a1 + u1
                return (a0, a1)

            a0, a1 = lax.fori_loop(0, L // UNROLL, body, (vzero, vzero))
            outbuf[row0 + q, pl.ds(0, LANES)] = a0 * SCALE
            outbuf[row0 + q, pl.ds(LANES, LANES)] = a1 * SCALE

    @pl.loop(0, NPANEL)
    def _(p):
        pltpu.sync_copy(tokens_hbm.at[pl.ds(tok0 + p * PANELTOK, PANELTOK)], idx_v)
        start_gather(0, buf0, sem0)

        @pl.loop(0, SPP, step=2)
        def _(g):
            start_gather(g + 1, buf1, sem1)
            wait_gather(buf0, sem0)
            accumulate(buf0, p * ROWS_PER_PANEL + g * (SW // L))

            @pl.when(g + 2 < SPP)
            def _():
                start_gather(g + 2, buf0, sem0)

            wait_gather(buf1, sem1)
            accumulate(buf1, p * ROWS_PER_PANEL + (g + 1) * (SW // L))

    pltpu.sync_copy(outbuf, out_hbm.at[pl.ds(out_row0, B_PER_TILE)])


@jax.jit
def kernel(tokens, table):
    tokens1d = tokens.astype(jnp.int32).reshape(TOK)
    perm = jnp.arange(D).reshape(2, D // 2).T.reshape(D)
    tableh = table[:, perm].astype(jnp.bfloat16)

    mesh = plsc.VectorSubcoreMesh(core_axis_name="c", subcore_axis_name="s")
    run = pl.kernel(
        _embed_body,
        out_type=jax.ShapeDtypeStruct((B, D), jnp.float32),
        mesh=mesh,
        compiler_params=pltpu.CompilerParams(
            use_tc_tiling_on_sc=False, needs_layout_passes=False
        ),
        scratch_types=[
            pltpu.VMEM((PANELTOK,), jnp.int32),
            pltpu.VMEM((SW, D), jnp.bfloat16),
            pltpu.VMEM((SW, D), jnp.bfloat16),
            pltpu.VMEM((B_PER_TILE, D), jnp.float32),
            pltpu.SemaphoreType.DMA,
            pltpu.SemaphoreType.DMA,
        ],
    )
    return run(tokens1d, tableh)


# speedup vs baseline: 1.4014x; 1.4014x over previous
import jax
import jax.numpy as jnp
from jax import lax
from jax.experimental import pallas as pl
from jax.experimental.pallas import tpu as pltpu
from jax.experimental.pallas import tpu_sc as plsc

VOCAB = 1000000
D = 32
B = 16384
L = 200
NC = 2
NS = 16
LANES = 16
NW = NC * NS
TOK = B * L
TOK_PER_TILE = TOK // NW
B_PER_TILE = B // NW
SW = 4 * L
SPP = 16
PANELTOK = SW * SPP
NPANEL = TOK_PER_TILE // PANELTOK
ROWS_PER_PANEL = PANELTOK // L
UNROLL = 8
SCALE = 1.0 / L


def _embed_body(tokens_hbm, table_hbm, out_hbm,
                idx0, idx1, buf0, buf1, outbuf, sem0, sem1, semi0, semi1):
    c = lax.axis_index("c")
    s = lax.axis_index("s")
    tile = c * NS + s
    tok0 = tile * TOK_PER_TILE
    out_row0 = tile * B_PER_TILE

    vzero = jnp.zeros((LANES,), jnp.float32)

    def load_panel(p, idx_v, semi):
        pltpu.async_copy(
            tokens_hbm.at[pl.ds(tok0 + p * PANELTOK, PANELTOK)], idx_v, semi)

    def wait_panel(idx_v, semi):
        pltpu.make_async_copy(
            tokens_hbm.at[pl.ds(0, PANELTOK)], idx_v, semi).wait()

    def start_gather(idx_v, g, buf, sem):
        pltpu.async_copy(table_hbm.at[idx_v.at[pl.ds(g * SW, SW)]], buf, sem)

    def wait_gather(idx_v, buf, sem):
        pltpu.make_async_copy(table_hbm.at[idx_v.at[pl.ds(0, SW)]], buf, sem).wait()

    def accumulate(buf, row0):
        for q in range(SW // L):
            def body(i, carry):
                a0, a1 = carry
                for u in range(UNROLL):
                    r = q * L + i * UNROLL + u
                    a0 = a0 + buf[r, pl.ds(0, LANES)]
                    a1 = a1 + buf[r, pl.ds(LANES, LANES)]
                return (a0, a1)

            a0, a1 = lax.fori_loop(0, L // UNROLL, body, (vzero, vzero))
            outbuf[row0 + q, pl.ds(0, LANES)] = a0 * SCALE
            outbuf[row0 + q, pl.ds(LANES, LANES)] = a1 * SCALE

    def process_panel(idx_v, p):
        start_gather(idx_v, 0, buf0, sem0)

        @pl.loop(0, SPP, step=2)
        def _(g):
            start_gather(idx_v, g + 1, buf1, sem1)
            wait_gather(idx_v, buf0, sem0)
            accumulate(buf0, p * ROWS_PER_PANEL + g * (SW // L))

            @pl.when(g + 2 < SPP)
            def _():
                start_gather(idx_v, g + 2, buf0, sem0)

            wait_gather(idx_v, buf1, sem1)
            accumulate(buf1, p * ROWS_PER_PANEL + (g + 1) * (SW // L))

    load_panel(0, idx0, semi0)

    @pl.loop(0, NPANEL, step=2)
    def _(p):
        load_panel(p + 1, idx1, semi1)
        wait_panel(idx0, semi0)
        process_panel(idx0, p)

        @pl.when(p + 2 < NPANEL)
        def _():
            load_panel(p + 2, idx0, semi0)

        wait_panel(idx1, semi1)
        process_panel(idx1, p + 1)

    pltpu.sync_copy(outbuf, out_hbm.at[pl.ds(out_row0, B_PER_TILE)])


@jax.jit
def kernel(tokens, table):
    tokens1d = tokens.astype(jnp.int32).reshape(TOK)

    mesh = plsc.VectorSubcoreMesh(core_axis_name="c", subcore_axis_name="s")
    run = pl.kernel(
        _embed_body,
        out_type=jax.ShapeDtypeStruct((B, D), jnp.float32),
        mesh=mesh,
        compiler_params=pltpu.CompilerParams(use_tc_tiling_on_sc=False),
        scratch_types=[
            pltpu.VMEM((PANELTOK,), jnp.int32),
            pltpu.VMEM((PANELTOK,), jnp.int32),
            pltpu.VMEM((SW, D), jnp.float32),
            pltpu.VMEM((SW, D), jnp.float32),
            pltpu.VMEM((B_PER_TILE, D), jnp.float32),
            pltpu.SemaphoreType.DMA,
            pltpu.SemaphoreType.DMA,
            pltpu.SemaphoreType.DMA,
            pltpu.SemaphoreType.DMA,
        ],
    )
    return run(tokens1d, table)


# speedup vs baseline: 1.4824x; 1.0577x over previous
import jax
import jax.numpy as jnp
from jax import lax
from jax.experimental import pallas as pl
from jax.experimental.pallas import tpu as pltpu
from jax.experimental.pallas import tpu_sc as plsc

VOCAB = 1000000
D = 32
B = 16384
L = 200
NC = 2
NS = 16
LANES = 16
NW = NC * NS
TOK = B * L
TOK_PER_TILE = TOK // NW
B_PER_TILE = B // NW
SW = 2 * L
SPP = 32
PANELTOK = SW * SPP
NPANEL = TOK_PER_TILE // PANELTOK
ROWS_PER_PANEL = PANELTOK // L
UNROLL = 8
SCALE = 1.0 / L


def _embed_body(tokens_hbm, table_hbm, out_hbm,
                idx0, idx1, buf0, buf1, buf2, buf3, outbuf,
                sem0, sem1, sem2, sem3, semi0, semi1):
    c = lax.axis_index("c")
    s = lax.axis_index("s")
    tile = c * NS + s
    tok0 = tile * TOK_PER_TILE
    out_row0 = tile * B_PER_TILE

    vzero = jnp.zeros((LANES,), jnp.float32)

    def load_panel(p, idx_v, semi):
        pltpu.async_copy(
            tokens_hbm.at[pl.ds(tok0 + p * PANELTOK, PANELTOK)], idx_v, semi)

    def wait_panel(idx_v, semi):
        pltpu.make_async_copy(
            tokens_hbm.at[pl.ds(0, PANELTOK)], idx_v, semi).wait()

    def start_gather(idx_v, g, buf, sem):
        pltpu.async_copy(table_hbm.at[idx_v.at[pl.ds(g * SW, SW)]], buf, sem)

    def wait_gather(idx_v, buf, sem):
        pltpu.make_async_copy(table_hbm.at[idx_v.at[pl.ds(0, SW)]], buf, sem).wait()

    def accumulate(buf, row0):
        for q in range(SW // L):
            def body(i, carry):
                a0, a1 = carry
                for u in range(UNROLL):
                    r = q * L + i * UNROLL + u
                    a0 = a0 + buf[r, pl.ds(0, LANES)]
                    a1 = a1 + buf[r, pl.ds(LANES, LANES)]
                return (a0, a1)

            a0, a1 = lax.fori_loop(0, L // UNROLL, body, (vzero, vzero))
            outbuf[row0 + q, pl.ds(0, LANES)] = a0 * SCALE
            outbuf[row0 + q, pl.ds(LANES, LANES)] = a1 * SCALE

    bufs = ((buf0, sem0), (buf1, sem1), (buf2, sem2), (buf3, sem3))

    def process_panel(idx_v, p):
        for k in range(3):
            start_gather(idx_v, k, *bufs[k])

        @pl.loop(0, SPP, step=4)
        def _(g):
            start_gather(idx_v, g + 3, *bufs[3])
            for k in range(4):
                buf, sem = bufs[k]
                wait_gather(idx_v, buf, sem)
                accumulate(buf, p * ROWS_PER_PANEL + (g + k) * (SW // L))
                if k < 3:
                    @pl.when(g + 4 + k < SPP)
                    def _(buf=buf, sem=sem, k=k):
                        start_gather(idx_v, g + 4 + k, buf, sem)

    load_panel(0, idx0, semi0)

    @pl.loop(0, NPANEL, step=2)
    def _(p):
        load_panel(p + 1, idx1, semi1)
        wait_panel(idx0, semi0)
        process_panel(idx0, p)

        @pl.when(p + 2 < NPANEL)
        def _():
            load_panel(p + 2, idx0, semi0)

        wait_panel(idx1, semi1)
        process_panel(idx1, p + 1)

    pltpu.sync_copy(outbuf, out_hbm.at[pl.ds(out_row0, B_PER_TILE)])


@jax.jit
def kernel(tokens, table):
    tokens1d = tokens.astype(jnp.int32).reshape(TOK)

    mesh = plsc.VectorSubcoreMesh(core_axis_name="c", subcore_axis_name="s")
    run = pl.kernel(
        _embed_body,
        out_type=jax.ShapeDtypeStruct((B, D), jnp.float32),
        mesh=mesh,
        compiler_params=pltpu.CompilerParams(use_tc_tiling_on_sc=False),
        scratch_types=[
            pltpu.VMEM((PANELTOK,), jnp.int32),
            pltpu.VMEM((PANELTOK,), jnp.int32),
            pltpu.VMEM((SW, D), jnp.float32),
            pltpu.VMEM((SW, D), jnp.float32),
            pltpu.VMEM((SW, D), jnp.float32),
            pltpu.VMEM((SW, D), jnp.float32),
            pltpu.VMEM((B_PER_TILE, D), jnp.float32),
            pltpu.SemaphoreType.DMA,
            pltpu.SemaphoreType.DMA,
            pltpu.SemaphoreType.DMA,
            pltpu.SemaphoreType.DMA,
            pltpu.SemaphoreType.DMA,
            pltpu.SemaphoreType.DMA,
        ],
    )
    return run(tokens1d, table)


# speedup vs baseline: 1.4996x; 1.0116x over previous
import jax
import jax.numpy as jnp
from jax import lax
from jax.experimental import pallas as pl
from jax.experimental.pallas import tpu as pltpu
from jax.experimental.pallas import tpu_sc as plsc

VOCAB = 1000000
D = 32
B = 16384
L = 200
NC = 2
NS = 16
LANES = 16
NW = NC * NS
TOK = B * L
TOK_PER_TILE = TOK // NW
B_PER_TILE = B // NW
SW = L
SPP = 64
PANELTOK = SW * SPP
NPANEL = TOK_PER_TILE // PANELTOK
ROWS_PER_PANEL = PANELTOK // L
UNROLL = 8
SCALE = 1.0 / L


NBUF = 8


def _embed_body(tokens_hbm, table_hbm, out_hbm,
                idx0, idx1, buf0, buf1, buf2, buf3, buf4, buf5, buf6, buf7,
                outbuf, sem0, sem1, sem2, sem3, sem4, sem5, sem6, sem7,
                semi0, semi1):
    c = lax.axis_index("c")
    s = lax.axis_index("s")
    tile = c * NS + s
    tok0 = tile * TOK_PER_TILE
    out_row0 = tile * B_PER_TILE

    vzero = jnp.zeros((LANES,), jnp.float32)

    def load_panel(p, idx_v, semi):
        pltpu.async_copy(
            tokens_hbm.at[pl.ds(tok0 + p * PANELTOK, PANELTOK)], idx_v, semi)

    def wait_panel(idx_v, semi):
        pltpu.make_async_copy(
            tokens_hbm.at[pl.ds(0, PANELTOK)], idx_v, semi).wait()

    def start_gather(idx_v, g, buf, sem):
        pltpu.async_copy(table_hbm.at[idx_v.at[pl.ds(g * SW, SW)]], buf, sem)

    def wait_gather(idx_v, buf, sem):
        pltpu.make_async_copy(table_hbm.at[idx_v.at[pl.ds(0, SW)]], buf, sem).wait()

    def accumulate(buf, row0):
        for q in range(SW // L):
            def body(i, carry):
                a0, a1 = carry
                for u in range(UNROLL):
                    r = q * L + i * UNROLL + u
                    a0 = a0 + buf[r, pl.ds(0, LANES)]
                    a1 = a1 + buf[r, pl.ds(LANES, LANES)]
                return (a0, a1)

            a0, a1 = lax.fori_loop(0, L // UNROLL, body, (vzero, vzero))
            outbuf[row0 + q, pl.ds(0, LANES)] = a0 * SCALE
            outbuf[row0 + q, pl.ds(LANES, LANES)] = a1 * SCALE

    bufs = ((buf0, sem0), (buf1, sem1), (buf2, sem2), (buf3, sem3),
            (buf4, sem4), (buf5, sem5), (buf6, sem6), (buf7, sem7))

    def process_panel(idx_v, p):
        for k in range(NBUF - 1):
            start_gather(idx_v, k, *bufs[k])

        @pl.loop(0, SPP, step=NBUF)
        def _(g):
            start_gather(idx_v, g + NBUF - 1, *bufs[NBUF - 1])
            for k in range(NBUF):
                buf, sem = bufs[k]
                wait_gather(idx_v, buf, sem)
                accumulate(buf, p * ROWS_PER_PANEL + (g + k) * (SW // L))
                if k < NBUF - 1:
                    @pl.when(g + NBUF + k < SPP)
                    def _(buf=buf, sem=sem, k=k):
                        start_gather(idx_v, g + NBUF + k, buf, sem)

    load_panel(0, idx0, semi0)

    @pl.loop(0, NPANEL, step=2)
    def _(p):
        load_panel(p + 1, idx1, semi1)
        wait_panel(idx0, semi0)
        process_panel(idx0, p)

        @pl.when(p + 2 < NPANEL)
        def _():
            load_panel(p + 2, idx0, semi0)

        wait_panel(idx1, semi1)
        process_panel(idx1, p + 1)

    pltpu.sync_copy(outbuf, out_hbm.at[pl.ds(out_row0, B_PER_TILE)])


@jax.jit
def kernel(tokens, table):
    tokens1d = tokens.astype(jnp.int32).reshape(TOK)

    mesh = plsc.VectorSubcoreMesh(core_axis_name="c", subcore_axis_name="s")
    run = pl.kernel(
        _embed_body,
        out_type=jax.ShapeDtypeStruct((B, D), jnp.float32),
        mesh=mesh,
        compiler_params=pltpu.CompilerParams(use_tc_tiling_on_sc=False),
        scratch_types=[
            pltpu.VMEM((PANELTOK,), jnp.int32),
            pltpu.VMEM((PANELTOK,), jnp.int32),
            *[pltpu.VMEM((SW, D), jnp.float32) for _ in range(8)],
            pltpu.VMEM((B_PER_TILE, D), jnp.float32),
            *[pltpu.SemaphoreType.DMA for _ in range(10)],
        ],
    )
    return run(tokens1d, table)
